# baseline (speedup 1.0000x reference)
import functools

import jax
import jax.numpy as jnp
from jax import lax
from jax.experimental import pallas as pl
from jax.experimental.pallas import tpu as pltpu
from jax.experimental.pallas import tpu_sc as plsc

N_NODES = 100000
N_EDGES = 1600000
NUM_ELEMENTS = 100
EMBED_DIM = 56
NODE_DIM = 128
NUM_BASIS = 20
CUTOFF = 8.0

_NC = 2
_NS = 16
_NW = _NC * _NS

_E_PER_W = N_EDGES // _NW
_ECHUNK = 2000
_ECHUNKS = _E_PER_W // _ECHUNK

_NCHUNK = 400
_NCHUNKS_TOTAL = N_NODES // _NCHUNK
_NFULL = _NCHUNKS_TOTAL // _NW
_NREM = _NCHUNKS_TOTAL - _NFULL * _NW


_NBLK = 800


def _node_body(at_ref, emb_ref, w_ref, b_ref, out_ref, nt_ref):
    @pl.when(pl.program_id(0) == 0)
    def _():
        nt_ref[...] = (
            jnp.dot(emb_ref[...], w_ref[...],
                    preferred_element_type=jnp.float32)
            + b_ref[...]
        )

    lanes = lax.broadcasted_iota(jnp.int32, (_NBLK, 128), 1)
    m = (at_ref[...] == lanes).astype(jnp.float32)
    out_ref[...] = jnp.dot(m, nt_ref[...], preferred_element_type=jnp.float32)


def _node_embed(at2d, embpad, W, b2):
    return pl.pallas_call(
        _node_body,
        grid=(N_NODES // _NBLK,),
        in_specs=[
            pl.BlockSpec((_NBLK, 1), lambda i: (i, 0)),
            pl.BlockSpec((128, EMBED_DIM), lambda i: (0, 0)),
            pl.BlockSpec((EMBED_DIM, NODE_DIM), lambda i: (0, 0)),
            pl.BlockSpec((1, NODE_DIM), lambda i: (0, 0)),
        ],
        out_specs=pl.BlockSpec((_NBLK, NODE_DIM), lambda i: (i, 0)),
        out_shape=jax.ShapeDtypeStruct((N_NODES, NODE_DIM), jnp.float32),
        scratch_shapes=[pltpu.VMEM((128, NODE_DIM), jnp.float32)],
    )(at2d, embpad, W, b2)


_SPLITB = 32768


def _split_body(ei_ref, s_ref, d_ref):
    e = ei_ref[...]
    s_ref[...] = e[0]
    d_ref[...] = e[1]


def _edge_split(edge_index):
    return pl.pallas_call(
        _split_body,
        grid=((N_EDGES + _SPLITB - 1) // _SPLITB,),
        in_specs=[pl.BlockSpec((2, _SPLITB), lambda i: (0, i))],
        out_specs=[
            pl.BlockSpec((_SPLITB,), lambda i: (i,)),
            pl.BlockSpec((_SPLITB,), lambda i: (i,)),
        ],
        out_shape=[
            jax.ShapeDtypeStruct((N_EDGES,), jnp.int32),
            jax.ShapeDtypeStruct((N_EDGES,), jnp.int32),
        ],
    )(edge_index)


def _sc_body(src_hbm, dst_hbm, posf_hbm,
             d2_hbm, fc_hbm,
             sidx_v, didx_v,
             s3x_v, s3y_v, s3z_v, d3x_v, d3y_v, d3z_v,
             sx_v, sy_v, sz_v, tx_v, ty_v, tz_v, d2_v, fc_v,
             spf,
             sem_a, sem_b):
    c = lax.axis_index("c")
    s = lax.axis_index("s")
    wid = s * _NC + c

    @pl.when(s == 0)
    def _stage():
        pltpu.sync_copy(posf_hbm, spf)

    plsc.subcore_barrier()

    ebase = wid * _E_PER_W

    def echunk(j, carry):
        base = ebase + j * _ECHUNK
        pltpu.sync_copy(src_hbm.at[pl.ds(base, _ECHUNK)], sidx_v)
        pltpu.sync_copy(dst_hbm.at[pl.ds(base, _ECHUNK)], didx_v)

        def scale(i, carry2):
            sl = pl.ds(i * 16, 16)
            s3 = sidx_v[sl] * 3
            d3 = didx_v[sl] * 3
            s3x_v[sl] = s3
            s3y_v[sl] = s3 + 1
            s3z_v[sl] = s3 + 2
            d3x_v[sl] = d3
            d3y_v[sl] = d3 + 1
            d3z_v[sl] = d3 + 2
            return carry2

        lax.fori_loop(0, _ECHUNK // 16, scale, 0)

        cps = [
            pltpu.async_copy(spf.at[s3x_v], sx_v, sem_a),
            pltpu.async_copy(spf.at[s3y_v], sy_v, sem_a),
            pltpu.async_copy(spf.at[s3z_v], sz_v, sem_a),
            pltpu.async_copy(spf.at[d3x_v], tx_v, sem_b),
            pltpu.async_copy(spf.at[d3y_v], ty_v, sem_b),
            pltpu.async_copy(spf.at[d3z_v], tz_v, sem_b),
        ]
        for cp in cps:
            cp.wait()

        def inner(i, carry2):
            sl = pl.ds(i * 16, 16)
            vx = sx_v[sl] - tx_v[sl]
            vy = sy_v[sl] - ty_v[sl]
            vz = sz_v[sl] - tz_v[sl]
            t = vx * vx + vy * vy + vz * vz
            d2_v[sl] = t
            e = jnp.exp(-t / (CUTOFF * CUTOFF - t))
            fc_v[sl] = jnp.where(t < CUTOFF * CUTOFF, e, 0.0)
            return carry2

        lax.fori_loop(0, _ECHUNK // 16, inner, 0)
        q = base // 16000
        rem = base - q * 16000
        pltpu.sync_copy(d2_v, d2_hbm.at[pl.ds(q * 16384 + rem, _ECHUNK)])
        pltpu.sync_copy(fc_v, fc_hbm.at[pl.ds(base, _ECHUNK)])
        return carry

    lax.fori_loop(0, _ECHUNKS, echunk, 0)


def _sc_gather(src, dst, posf):
    mesh = plsc.VectorSubcoreMesh(core_axis_name="c", subcore_axis_name="s")
    f = functools.partial(
        pl.kernel,
        mesh=mesh,
        out_type=[
            jax.ShapeDtypeStruct((100 * 16384,), jnp.float32),
            jax.ShapeDtypeStruct((N_EDGES,), jnp.float32),
        ],
        scratch_types=[
            pltpu.VMEM((_ECHUNK,), jnp.int32),
            pltpu.VMEM((_ECHUNK,), jnp.int32),
            pltpu.VMEM((_ECHUNK,), jnp.int32),
            pltpu.VMEM((_ECHUNK,), jnp.int32),
            pltpu.VMEM((_ECHUNK,), jnp.int32),
            pltpu.VMEM((_ECHUNK,), jnp.int32),
            pltpu.VMEM((_ECHUNK,), jnp.int32),
            pltpu.VMEM((_ECHUNK,), jnp.int32),
            pltpu.VMEM((_ECHUNK,), jnp.float32),
            pltpu.VMEM((_ECHUNK,), jnp.float32),
            pltpu.VMEM((_ECHUNK,), jnp.float32),
            pltpu.VMEM((_ECHUNK,), jnp.float32),
            pltpu.VMEM((_ECHUNK,), jnp.float32),
            pltpu.VMEM((_ECHUNK,), jnp.float32),
            pltpu.VMEM((_ECHUNK,), jnp.float32),
            pltpu.VMEM((_ECHUNK,), jnp.float32),
            pltpu.VMEM_SHARED((3 * N_NODES,), jnp.float32),
            pltpu.SemaphoreType.DMA,
            pltpu.SemaphoreType.DMA,
        ],
    )(_sc_body)
    return f(src, dst, posf)


_R = 125
_QROWS = N_EDGES // 128
_GRID = _QROWS // _R
_NPAD = 24


def _edge_body(d2_ref, rbf_ref):
    d2 = d2_ref[0][:_R]
    dist = jnp.sqrt(d2)
    safe = jnp.maximum(dist, 1e-8)
    theta = safe * (jnp.pi / CUTOFF)
    s1 = jnp.sin(theta)
    c1 = jnp.cos(theta)

    inside = dist < CUTOFF
    dcl = jnp.where(inside, dist, 0.0)
    val = jnp.exp(-(dcl * dcl) / ((CUTOFF - dcl) * (CUTOFF + dcl)))
    fc = jnp.where(inside, val, 0.0)

    g = 0.5 * fc / safe
    twoc = 2.0 * c1

    rows = []
    s_nm2 = jnp.zeros_like(s1)
    s_nm1 = s1
    rows.append(g * s_nm1)
    for _ in range(NUM_BASIS - 1):
        s_n = twoc * s_nm1 - s_nm2
        rows.append(g * s_n)
        s_nm2, s_nm1 = s_nm1, s_n
    zero = jnp.zeros_like(s1)
    while len(rows) < _NPAD:
        rows.append(zero)

    aa = jnp.stack(rows, axis=0)

    r = lax.broadcasted_iota(jnp.int32, (_NPAD, _NPAD), 0)
    cidx = lax.broadcasted_iota(jnp.int32, (_NPAD, _NPAD), 1)
    eye = jnp.where(r == cidx, 1.0, 0.0).astype(jnp.float32)

    t = lax.dot_general(
        aa, eye, (((0,), (0,)), ((), ())),
        preferred_element_type=jnp.float32,
    )
    rbf_ref[0] = t[:, :, :NUM_BASIS].reshape(_R, 128 * NUM_BASIS)


def _edge_math(d2q):
    return pl.pallas_call(
        _edge_body,
        grid=(_GRID,),
        in_specs=[pl.BlockSpec((1, 128, 128), lambda i: (i, 0, 0))],
        out_specs=pl.BlockSpec((1, _R, 128 * NUM_BASIS), lambda i: (i, 0, 0)),
        out_shape=jax.ShapeDtypeStruct((_GRID, _R, 128 * NUM_BASIS),
                                       jnp.float32),
    )(d2q)


def kernel(at_no, pos, edge_index, embed_table, W, b):
    posf = pos.reshape(3 * N_NODES)
    src, dst = _edge_split(edge_index)
    embpad = jnp.pad(embed_table, ((0, 128 - NUM_ELEMENTS), (0, 0)))
    node = _node_embed(at_no.reshape(N_NODES, 1), embpad, W,
                       b.reshape(1, NODE_DIM))
    d2p, fc = _sc_gather(src, dst, posf)
    rbf = _edge_math(d2p.reshape(_GRID, 128, 128))
    return (node, rbf.reshape(N_EDGES, NUM_BASIS), fc.reshape(N_EDGES, 1))


# speedup vs baseline: 2.7527x; 2.7527x over previous
import functools

import jax
import jax.numpy as jnp
from jax import lax
from jax.experimental import pallas as pl
from jax.experimental.pallas import tpu as pltpu
from jax.experimental.pallas import tpu_sc as plsc

N_NODES = 100000
N_EDGES = 1600000
NUM_ELEMENTS = 100
EMBED_DIM = 56
NODE_DIM = 128
NUM_BASIS = 20
CUTOFF = 8.0

_NC = 2
_NS = 16
_NW = _NC * _NS

_E_PER_W = N_EDGES // _NW
_ECHUNK = 2000
_ECHUNKS = _E_PER_W // _ECHUNK

_NCHUNK = 400
_NCHUNKS_TOTAL = N_NODES // _NCHUNK
_NFULL = _NCHUNKS_TOTAL // _NW
_NREM = _NCHUNKS_TOTAL - _NFULL * _NW


_NBLK = 1024


def _node_body(at_ref, emb_ref, w_ref, b_ref, out_ref, nt_ref):
    @pl.when(pl.program_id(0) == 0)
    def _():
        nt_ref[...] = (
            jnp.dot(emb_ref[...], w_ref[...],
                    preferred_element_type=jnp.float32)
            + b_ref[...]
        )

    a = at_ref[...]
    rows = lax.broadcasted_iota(jnp.int32, (128, _NBLK), 0)
    m = (rows == a[None, :]).astype(jnp.float32)
    out_ref[...] = lax.dot_general(
        m, nt_ref[...], (((0,), (0,)), ((), ())),
        preferred_element_type=jnp.float32,
    )


def _node_embed(at_no, embpad, W, b2):
    return pl.pallas_call(
        _node_body,
        grid=((N_NODES + _NBLK - 1) // _NBLK,),
        in_specs=[
            pl.BlockSpec((_NBLK,), lambda i: (i,)),
            pl.BlockSpec((128, EMBED_DIM), lambda i: (0, 0)),
            pl.BlockSpec((EMBED_DIM, NODE_DIM), lambda i: (0, 0)),
            pl.BlockSpec((1, NODE_DIM), lambda i: (0, 0)),
        ],
        out_specs=pl.BlockSpec((_NBLK, NODE_DIM), lambda i: (i, 0)),
        out_shape=jax.ShapeDtypeStruct((N_NODES, NODE_DIM), jnp.float32),
        scratch_shapes=[pltpu.VMEM((128, NODE_DIM), jnp.float32)],
    )(at_no, embpad, W, b2)


_SPLITB = 32768


def _split_body(ei_ref, s_ref, d_ref):
    e = ei_ref[...]
    s_ref[...] = e[0]
    d_ref[...] = e[1]


def _edge_split(edge_index):
    return pl.pallas_call(
        _split_body,
        grid=((N_EDGES + _SPLITB - 1) // _SPLITB,),
        in_specs=[pl.BlockSpec((2, _SPLITB), lambda i: (0, i))],
        out_specs=[
            pl.BlockSpec((_SPLITB,), lambda i: (i,)),
            pl.BlockSpec((_SPLITB,), lambda i: (i,)),
        ],
        out_shape=[
            jax.ShapeDtypeStruct((N_EDGES,), jnp.int32),
            jax.ShapeDtypeStruct((N_EDGES,), jnp.int32),
        ],
    )(edge_index)


def _sc_body(src_hbm, dst_hbm, posf_hbm,
             d2_hbm, fc_hbm,
             sidx_v, didx_v,
             s3x_v, s3y_v, s3z_v, d3x_v, d3y_v, d3z_v,
             sx_v, sy_v, sz_v, tx_v, ty_v, tz_v, d2_v, fc_v,
             spf,
             sem_a, sem_b):
    c = lax.axis_index("c")
    s = lax.axis_index("s")
    wid = s * _NC + c

    @pl.when(s == 0)
    def _stage():
        pltpu.sync_copy(posf_hbm, spf)

    plsc.subcore_barrier()

    ebase = wid * _E_PER_W

    def echunk(j, carry):
        base = ebase + j * _ECHUNK
        pltpu.sync_copy(src_hbm.at[pl.ds(base, _ECHUNK)], sidx_v)
        pltpu.sync_copy(dst_hbm.at[pl.ds(base, _ECHUNK)], didx_v)

        def scale(i, carry2):
            sl = pl.ds(i * 16, 16)
            s3 = sidx_v[sl] * 3
            d3 = didx_v[sl] * 3
            s3x_v[sl] = s3
            s3y_v[sl] = s3 + 1
            s3z_v[sl] = s3 + 2
            d3x_v[sl] = d3
            d3y_v[sl] = d3 + 1
            d3z_v[sl] = d3 + 2
            return carry2

        lax.fori_loop(0, _ECHUNK // 16, scale, 0)

        cps = [
            pltpu.async_copy(spf.at[s3x_v], sx_v, sem_a),
            pltpu.async_copy(spf.at[s3y_v], sy_v, sem_a),
            pltpu.async_copy(spf.at[s3z_v], sz_v, sem_a),
            pltpu.async_copy(spf.at[d3x_v], tx_v, sem_b),
            pltpu.async_copy(spf.at[d3y_v], ty_v, sem_b),
            pltpu.async_copy(spf.at[d3z_v], tz_v, sem_b),
        ]
        for cp in cps:
            cp.wait()

        def inner(i, carry2):
            sl = pl.ds(i * 16, 16)
            vx = sx_v[sl] - tx_v[sl]
            vy = sy_v[sl] - ty_v[sl]
            vz = sz_v[sl] - tz_v[sl]
            t = vx * vx + vy * vy + vz * vz
            d2_v[sl] = t
            e = jnp.exp(-t / (CUTOFF * CUTOFF - t))
            fc_v[sl] = jnp.where(t < CUTOFF * CUTOFF, e, 0.0)
            return carry2

        lax.fori_loop(0, _ECHUNK // 16, inner, 0)
        q = base // 16000
        rem = base - q * 16000
        pltpu.sync_copy(d2_v, d2_hbm.at[pl.ds(q * 16384 + rem, _ECHUNK)])
        pltpu.sync_copy(fc_v, fc_hbm.at[pl.ds(base, _ECHUNK)])
        return carry

    lax.fori_loop(0, _ECHUNKS, echunk, 0)


def _sc_gather(src, dst, posf):
    mesh = plsc.VectorSubcoreMesh(core_axis_name="c", subcore_axis_name="s")
    f = functools.partial(
        pl.kernel,
        mesh=mesh,
        out_type=[
            jax.ShapeDtypeStruct((100 * 16384,), jnp.float32),
            jax.ShapeDtypeStruct((N_EDGES,), jnp.float32),
        ],
        scratch_types=[
            pltpu.VMEM((_ECHUNK,), jnp.int32),
            pltpu.VMEM((_ECHUNK,), jnp.int32),
            pltpu.VMEM((_ECHUNK,), jnp.int32),
            pltpu.VMEM((_ECHUNK,), jnp.int32),
            pltpu.VMEM((_ECHUNK,), jnp.int32),
            pltpu.VMEM((_ECHUNK,), jnp.int32),
            pltpu.VMEM((_ECHUNK,), jnp.int32),
            pltpu.VMEM((_ECHUNK,), jnp.int32),
            pltpu.VMEM((_ECHUNK,), jnp.float32),
            pltpu.VMEM((_ECHUNK,), jnp.float32),
            pltpu.VMEM((_ECHUNK,), jnp.float32),
            pltpu.VMEM((_ECHUNK,), jnp.float32),
            pltpu.VMEM((_ECHUNK,), jnp.float32),
            pltpu.VMEM((_ECHUNK,), jnp.float32),
            pltpu.VMEM((_ECHUNK,), jnp.float32),
            pltpu.VMEM((_ECHUNK,), jnp.float32),
            pltpu.VMEM_SHARED((3 * N_NODES,), jnp.float32),
            pltpu.SemaphoreType.DMA,
            pltpu.SemaphoreType.DMA,
        ],
    )(_sc_body)
    return f(src, dst, posf)


_R = 125
_QROWS = N_EDGES // 128
_GRID = _QROWS // _R
_NPAD = 24


def _edge_body(d2_ref, rbf_ref):
    d2 = d2_ref[0][:_R]
    dist = jnp.sqrt(d2)
    safe = jnp.maximum(dist, 1e-8)
    theta = safe * (jnp.pi / CUTOFF)
    s1 = jnp.sin(theta)
    c1 = jnp.cos(theta)

    inside = dist < CUTOFF
    dcl = jnp.where(inside, dist, 0.0)
    val = jnp.exp(-(dcl * dcl) / ((CUTOFF - dcl) * (CUTOFF + dcl)))
    fc = jnp.where(inside, val, 0.0)

    g = 0.5 * fc / safe
    twoc = 2.0 * c1

    rows = []
    s_nm2 = jnp.zeros_like(s1)
    s_nm1 = s1
    rows.append(g * s_nm1)
    for _ in range(NUM_BASIS - 1):
        s_n = twoc * s_nm1 - s_nm2
        rows.append(g * s_n)
        s_nm2, s_nm1 = s_nm1, s_n
    zero = jnp.zeros_like(s1)
    while len(rows) < _NPAD:
        rows.append(zero)

    aa = jnp.stack(rows, axis=0)

    r = lax.broadcasted_iota(jnp.int32, (_NPAD, _NPAD), 0)
    cidx = lax.broadcasted_iota(jnp.int32, (_NPAD, _NPAD), 1)
    eye = jnp.where(r == cidx, 1.0, 0.0).astype(jnp.float32)

    t = lax.dot_general(
        aa, eye, (((0,), (0,)), ((), ())),
        preferred_element_type=jnp.float32,
    )
    rbf_ref[...] = t[:, :, :NUM_BASIS].reshape(_R * 128, NUM_BASIS)


def _edge_math(d2q):
    return pl.pallas_call(
        _edge_body,
        grid=(_GRID,),
        in_specs=[pl.BlockSpec((1, 128, 128), lambda i: (i, 0, 0))],
        out_specs=pl.BlockSpec((_R * 128, NUM_BASIS), lambda i: (i, 0)),
        out_shape=jax.ShapeDtypeStruct((N_EDGES, NUM_BASIS), jnp.float32),
    )(d2q)


def kernel(at_no, pos, edge_index, embed_table, W, b):
    posf = pos.reshape(3 * N_NODES)
    src, dst = _edge_split(edge_index)
    embpad = jnp.pad(embed_table, ((0, 128 - NUM_ELEMENTS), (0, 0)))
    node = _node_embed(at_no, embpad, W, b.reshape(1, NODE_DIM))
    d2p, fc = _sc_gather(src, dst, posf)
    rbf = _edge_math(d2p.reshape(_GRID, 128, 128))
    return (node, rbf, fc.reshape(N_EDGES, 1))
